# Initial kernel scaffold; baseline (speedup 1.0000x reference)
import jax
import jax.numpy as jnp
from jax.experimental import pallas as pl


def kernel(x, s0_wa, s0_ba, s0_wb, s0_bb, s0_sc, s0_sh, s1_wa, s1_ba, s1_wb, s1_bb, s1_sc, s1_sh, s2_wa, s2_ba, s2_wb, s2_bb, s2_sc, s2_sh, s3_wa, s3_ba, s3_wb, s3_bb, s3_sc, s3_sh, fc1_w, fc1_b, fc2_w, fc2_b):
    raise NotImplementedError("write your pallas kernel here")


# speedup vs baseline: 4.9443x; 4.9443x over previous
import functools

import jax
import jax.numpy as jnp
from jax.experimental import pallas as pl
from jax.experimental.pallas import tpu as pltpu


def _round8(n):
    return (n + 7) & ~7


def _stage_kernel(H, x_ref, wa_ref, ba_ref, wb_ref, bb_ref, sc_ref, sh_ref,
                  o_ref, mid_ref):
    Hp = H + 2
    U = H // 2
    Upt = x_ref.shape[2]
    cin2 = x_ref.shape[3]
    cmid2 = mid_ref.shape[2]
    cout = o_ref.shape[3]
    cout2 = 2 * cout

    def conv(flat_in, w_ref, b_ref, co2):
        p = jnp.dot(flat_in, w_ref[...], preferred_element_type=jnp.float32)
        p = p.reshape(Hp, Upt, 9 * co2)
        acc = None
        for t in range(9):
            dy, dd = divmod(t, 3)
            term = p[dy:dy + H, dd:dd + U, t * co2:(t + 1) * co2]
            acc = term if acc is None else acc + term
        return jnp.maximum(acc + b_ref[...].reshape(1, 1, co2), 0.0)

    y = conv(x_ref[0].reshape(Hp * Upt, cin2), wa_ref, ba_ref, cmid2)

    mid_ref[0:1] = jnp.zeros((1, Upt, cmid2), jnp.float32)
    mid_ref[Hp - 1:Hp] = jnp.zeros((1, Upt, cmid2), jnp.float32)
    mid_ref[:, 0:1, :] = jnp.zeros((Hp, 1, cmid2), jnp.float32)
    mid_ref[:, U + 1:Upt, :] = jnp.zeros((Hp, Upt - U - 1, cmid2), jnp.float32)
    mid_ref[1:1 + H, 1:1 + U, :] = y

    y2 = conv(mid_ref[...].reshape(Hp * Upt, cmid2), wb_ref, bb_ref, cout2)

    y2r = y2.reshape(H // 2, 2, U, cout2)
    zh = jnp.maximum(y2r[:, 0], y2r[:, 1])
    z = jnp.maximum(zh[:, :, :cout], zh[:, :, cout:])
    o_ref[0] = z * sc_ref[...].reshape(1, 1, cout) + sh_ref[...].reshape(1, 1, cout)


def _const_spec(a):
    nd = a.ndim
    return pl.BlockSpec(a.shape, lambda i, _nd=nd: (0,) * _nd)


def _run_stage(xpad, wa2, ba2, wb2, bb2, sc, sh, H):
    n = xpad.shape[0]
    hp, upt, cin2 = xpad.shape[1], xpad.shape[2], xpad.shape[3]
    cmid2 = wa2.shape[1] // 9
    cout = wb2.shape[1] // 18
    h2 = H // 2
    u = H // 2

    kern = functools.partial(_stage_kernel, H)
    return pl.pallas_call(
        kern,
        out_shape=jax.ShapeDtypeStruct((n, h2, u, cout), jnp.float32),
        grid=(n,),
        in_specs=[pl.BlockSpec((1, hp, upt, cin2), lambda i: (i, 0, 0, 0)),
                  _const_spec(wa2), _const_spec(ba2), _const_spec(wb2),
                  _const_spec(bb2), _const_spec(sc), _const_spec(sh)],
        out_specs=pl.BlockSpec((1, h2, u, cout), lambda i: (i, 0, 0, 0)),
        scratch_shapes=[pltpu.VMEM((hp, upt, cmid2), jnp.float32)],
        compiler_params=pltpu.CompilerParams(
            dimension_semantics=("parallel",),
            vmem_limit_bytes=100 * 1024 * 1024,
        ),
    )(xpad, wa2, ba2, wb2, bb2, sc, sh)


def _classifier_kernel(x_ref, w1_ref, b1_ref, w2_ref, b2_ref, o_ref):
    h = jnp.dot(x_ref[...], w1_ref[...], preferred_element_type=jnp.float32)
    h = jnp.maximum(h + b1_ref[...], 0.0)
    o_ref[...] = jnp.dot(h, w2_ref[...],
                         preferred_element_type=jnp.float32) + b2_ref[...]


def _pack_w(wk):
    cin, cout = wk.shape[1], wk.shape[2]
    w = jnp.zeros((2 * cin, 9, 2 * cout), jnp.float32)
    for dy in range(3):
        for dlt in (-1, 0, 1):
            t = dy * 3 + dlt + 1
            for jin in range(2):
                for jout in range(2):
                    dx = 2 * dlt + jin - jout
                    if -1 <= dx <= 1:
                        w = w.at[jin * cin:(jin + 1) * cin, t,
                                 jout * cout:(jout + 1) * cout].set(
                                     wk[dy * 3 + dx + 1])
    return w.reshape(2 * cin, 18 * cout)


def kernel(x, s0_wa, s0_ba, s0_wb, s0_bb, s0_sc, s0_sh,
           s1_wa, s1_ba, s1_wb, s1_bb, s1_sc, s1_sh,
           s2_wa, s2_ba, s2_wb, s2_bb, s2_sc, s2_sh,
           s3_wa, s3_ba, s3_wb, s3_bb, s3_sc, s3_sh,
           fc1_w, fc1_b, fc2_w, fc2_b):
    n = x.shape[0]
    stages = [
        (s0_wa, s0_ba, s0_wb, s0_bb, s0_sc, s0_sh),
        (s1_wa, s1_ba, s1_wb, s1_bb, s1_sc, s1_sh),
        (s2_wa, s2_ba, s2_wb, s2_bb, s2_sc, s2_sh),
        (s3_wa, s3_ba, s3_wb, s3_bb, s3_sc, s3_sh),
    ]

    cur = jnp.transpose(x, (0, 2, 3, 1)).astype(jnp.float32)
    H = cur.shape[1]
    for wa, ba, wb, bb, sc, sh in stages:
        c = cur.shape[3]
        u = H // 2
        upt = _round8(u + 2)
        packed = cur.reshape(n, H, u, 2 * c)
        xpad = jnp.pad(packed, ((0, 0), (1, 1), (1, upt - u - 1), (0, 0)))
        cur = _run_stage(xpad, _pack_w(wa), jnp.tile(ba, (1, 2)),
                         _pack_w(wb), jnp.tile(bb, (1, 2)), sc, sh, H)
        H //= 2

    feats = cur.reshape(n, -1)
    out_c = fc2_w.shape[1]
    return pl.pallas_call(
        _classifier_kernel,
        out_shape=jax.ShapeDtypeStruct((n, out_c), jnp.float32),
    )(feats, fc1_w, fc1_b, fc2_w, fc2_b)


# speedup vs baseline: 7.2828x; 1.4730x over previous
import functools

import jax
import jax.numpy as jnp
from jax.experimental import pallas as pl
from jax.experimental.pallas import tpu as pltpu


def _round8(n):
    return (n + 7) & ~7


def _stage_kernel(H, x_ref, wa_ref, ba_ref, wb_ref, bb_ref, sc_ref, sh_ref,
                  o_ref, x3a_ref, x3b_ref):
    Hp = H + 2
    U = H // 2
    Upt = x_ref.shape[2]
    cin2 = x_ref.shape[3]
    cmid2 = x3b_ref.shape[2] // 3
    cout = o_ref.shape[3]
    cout2 = 2 * cout

    def conv(x3_ref, w_ref, b_ref, co2):
        flat = x3_ref[...].reshape(Hp * Upt, x3_ref.shape[2])
        p = jnp.dot(flat, w_ref[...], preferred_element_type=jnp.float32)
        p = p.reshape(Hp, Upt, 3 * co2)
        acc = p[0:H, 0:U, 0:co2]
        for dy in (1, 2):
            acc = acc + p[dy:dy + H, 0:U, dy * co2:(dy + 1) * co2]
        return jnp.maximum(acc + b_ref[...].reshape(1, 1, co2), 0.0)

    for b in range(3):
        x3a_ref[:, 0:U, b * cin2:(b + 1) * cin2] = x_ref[0, :, b:b + U, :]

    y = conv(x3a_ref, wa_ref, ba_ref, cmid2)

    x3b_ref[0:1] = jnp.zeros((1, Upt, 3 * cmid2), jnp.float32)
    x3b_ref[Hp - 1:Hp] = jnp.zeros((1, Upt, 3 * cmid2), jnp.float32)
    x3b_ref[:, 0:1, :] = jnp.zeros((Hp, 1, 3 * cmid2), jnp.float32)
    x3b_ref[:, U - 1:U, :] = jnp.zeros((Hp, 1, 3 * cmid2), jnp.float32)
    x3b_ref[1:H + 1, 1:U + 1, 0:cmid2] = y
    x3b_ref[1:H + 1, 0:U, cmid2:2 * cmid2] = y
    x3b_ref[1:H + 1, 0:U - 1, 2 * cmid2:3 * cmid2] = y[:, 1:U, :]

    y2 = conv(x3b_ref, wb_ref, bb_ref, cout2)

    y2r = y2.reshape(H // 2, 2, U, cout2)
    zh = jnp.maximum(y2r[:, 0], y2r[:, 1])
    z = jnp.maximum(zh[:, :, :cout], zh[:, :, cout:])
    o_ref[0] = z * sc_ref[...].reshape(1, 1, cout) + sh_ref[...].reshape(1, 1, cout)


def _const_spec(a):
    nd = a.ndim
    return pl.BlockSpec(a.shape, lambda i, _nd=nd: (0,) * _nd)


def _run_stage(xpad, wa2, ba2, wb2, bb2, sc, sh, H):
    n = xpad.shape[0]
    hp, upt, cin2 = xpad.shape[1], xpad.shape[2], xpad.shape[3]
    cmid2 = wa2.shape[1] // 3
    cout = wb2.shape[1] // 6
    h2 = H // 2
    u = H // 2

    kern = functools.partial(_stage_kernel, H)
    return pl.pallas_call(
        kern,
        out_shape=jax.ShapeDtypeStruct((n, h2, u, cout), jnp.float32),
        grid=(n,),
        in_specs=[pl.BlockSpec((1, hp, upt, cin2), lambda i: (i, 0, 0, 0)),
                  _const_spec(wa2), _const_spec(ba2), _const_spec(wb2),
                  _const_spec(bb2), _const_spec(sc), _const_spec(sh)],
        out_specs=pl.BlockSpec((1, h2, u, cout), lambda i: (i, 0, 0, 0)),
        scratch_shapes=[pltpu.VMEM((hp, upt, 3 * cin2), jnp.float32),
                        pltpu.VMEM((hp, upt, 3 * cmid2), jnp.float32)],
        compiler_params=pltpu.CompilerParams(
            dimension_semantics=("parallel",),
            vmem_limit_bytes=100 * 1024 * 1024,
        ),
    )(xpad, wa2, ba2, wb2, bb2, sc, sh)


def _classifier_kernel(x_ref, w1_ref, b1_ref, w2_ref, b2_ref, o_ref):
    h = jnp.dot(x_ref[...], w1_ref[...], preferred_element_type=jnp.float32)
    h = jnp.maximum(h + b1_ref[...], 0.0)
    o_ref[...] = jnp.dot(h, w2_ref[...],
                         preferred_element_type=jnp.float32) + b2_ref[...]


def _pack_w(wk):
    cin, cout = wk.shape[1], wk.shape[2]
    w = jnp.zeros((2 * cin, 9, 2 * cout), jnp.float32)
    for dy in range(3):
        for b in range(3):
            t = dy * 3 + b
            for jin in range(2):
                for jout in range(2):
                    dx = 2 * (b - 1) + jin - jout
                    if -1 <= dx <= 1:
                        w = w.at[jin * cin:(jin + 1) * cin, t,
                                 jout * cout:(jout + 1) * cout].set(
                                     wk[dy * 3 + dx + 1])
    w = w.reshape(2 * cin, 3, 3, 2 * cout).transpose(2, 0, 1, 3)
    return w.reshape(3 * 2 * cin, 3 * 2 * cout)


def kernel(x, s0_wa, s0_ba, s0_wb, s0_bb, s0_sc, s0_sh,
           s1_wa, s1_ba, s1_wb, s1_bb, s1_sc, s1_sh,
           s2_wa, s2_ba, s2_wb, s2_bb, s2_sc, s2_sh,
           s3_wa, s3_ba, s3_wb, s3_bb, s3_sc, s3_sh,
           fc1_w, fc1_b, fc2_w, fc2_b):
    n = x.shape[0]
    stages = [
        (s0_wa, s0_ba, s0_wb, s0_bb, s0_sc, s0_sh),
        (s1_wa, s1_ba, s1_wb, s1_bb, s1_sc, s1_sh),
        (s2_wa, s2_ba, s2_wb, s2_bb, s2_sc, s2_sh),
        (s3_wa, s3_ba, s3_wb, s3_bb, s3_sc, s3_sh),
    ]

    cur = jnp.transpose(x, (0, 2, 3, 1)).astype(jnp.float32)
    H = cur.shape[1]
    for wa, ba, wb, bb, sc, sh in stages:
        c = cur.shape[3]
        u = H // 2
        upt = _round8(u + 2)
        packed = cur.reshape(n, H, u, 2 * c)
        xpad = jnp.pad(packed, ((0, 0), (1, 1), (1, upt - u - 1), (0, 0)))
        cur = _run_stage(xpad, _pack_w(wa), jnp.tile(ba, (1, 2)),
                         _pack_w(wb), jnp.tile(bb, (1, 2)), sc, sh, H)
        H //= 2

    feats = cur.reshape(n, -1)
    out_c = fc2_w.shape[1]
    return pl.pallas_call(
        _classifier_kernel,
        out_shape=jax.ShapeDtypeStruct((n, out_c), jnp.float32),
    )(feats, fc1_w, fc1_b, fc2_w, fc2_b)


# speedup vs baseline: 9.2205x; 1.2661x over previous
import functools

import jax
import jax.numpy as jnp
from jax.experimental import pallas as pl
from jax.experimental.pallas import tpu as pltpu

_PACK = (8, 2, 2, 2)


def _round8(n):
    return (n + 7) & ~7


def _stage_kernel(H, p, x3_ref, wa_ref, ba_ref, wb_ref, bb_ref, sc_ref,
                  sh_ref, o_ref, x3b_ref):
    Hp = H + 2
    U = H // p
    Upt = x3_ref.shape[2]
    cmid_p = x3b_ref.shape[2] // 3
    cout_g = o_ref.shape[3]
    cout_p = 2 * cout_g

    def conv(x3_flat, w_ref, b_ref, co):
        q = jnp.dot(x3_flat, w_ref[...], preferred_element_type=jnp.float32)
        q = q.reshape(Hp, Upt, 3 * co)
        acc = q[0:H, 0:U, 0:co]
        for dy in (1, 2):
            acc = acc + q[dy:dy + H, 0:U, dy * co:(dy + 1) * co]
        return jnp.maximum(acc + b_ref[...].reshape(1, 1, co), 0.0)

    y = conv(x3_ref[0].reshape(Hp * Upt, x3_ref.shape[3]), wa_ref, ba_ref,
             cmid_p)

    x3b_ref[0:1] = jnp.zeros((1, Upt, 3 * cmid_p), jnp.float32)
    x3b_ref[Hp - 1:Hp] = jnp.zeros((1, Upt, 3 * cmid_p), jnp.float32)
    x3b_ref[:, 0:1, :] = jnp.zeros((Hp, 1, 3 * cmid_p), jnp.float32)
    x3b_ref[:, U - 1:U, :] = jnp.zeros((Hp, 1, 3 * cmid_p), jnp.float32)
    x3b_ref[1:H + 1, 1:U + 1, 0:cmid_p] = y
    x3b_ref[1:H + 1, 0:U, cmid_p:2 * cmid_p] = y
    x3b_ref[1:H + 1, 0:U - 1, 2 * cmid_p:3 * cmid_p] = y[:, 1:U, :]

    y2 = conv(x3b_ref[...].reshape(Hp * Upt, 3 * cmid_p), wb_ref, bb_ref,
              cout_p)

    y2r = y2.reshape(H // 2, 2, U, cout_p)
    zh = jnp.maximum(y2r[:, 0], y2r[:, 1])
    c1 = cout_p // p
    parts = []
    for k in range(p // 2):
        parts.append(jnp.maximum(zh[:, :, (2 * k) * c1:(2 * k + 1) * c1],
                                 zh[:, :, (2 * k + 1) * c1:(2 * k + 2) * c1]))
    z = parts[0] if len(parts) == 1 else jnp.concatenate(parts, axis=-1)
    o_ref[0] = z * sc_ref[...].reshape(1, 1, cout_g) + \
        sh_ref[...].reshape(1, 1, cout_g)


def _const_spec(a):
    nd = a.ndim
    return pl.BlockSpec(a.shape, lambda i, _nd=nd: (0,) * _nd)


def _run_stage(x3, wa2, ba2, wb2, bb2, sc, sh, H, p):
    n = x3.shape[0]
    hp, upt, k3 = x3.shape[1], x3.shape[2], x3.shape[3]
    cmid_p = wa2.shape[1] // 3
    cout_g = wb2.shape[1] // 6
    h2 = H // 2
    u = H // p

    kern = functools.partial(_stage_kernel, H, p)
    return pl.pallas_call(
        kern,
        out_shape=jax.ShapeDtypeStruct((n, h2, u, cout_g), jnp.float32),
        grid=(n,),
        in_specs=[pl.BlockSpec((1, hp, upt, k3), lambda i: (i, 0, 0, 0)),
                  _const_spec(wa2), _const_spec(ba2), _const_spec(wb2),
                  _const_spec(bb2), _const_spec(sc), _const_spec(sh)],
        out_specs=pl.BlockSpec((1, h2, u, cout_g), lambda i: (i, 0, 0, 0)),
        scratch_shapes=[pltpu.VMEM((hp, upt, 3 * cmid_p), jnp.float32)],
        compiler_params=pltpu.CompilerParams(
            dimension_semantics=("parallel",),
            vmem_limit_bytes=100 * 1024 * 1024,
        ),
    )(x3, wa2, ba2, wb2, bb2, sc, sh)


def _classifier_kernel(x_ref, w1_ref, b1_ref, w2_ref, b2_ref, o_ref):
    h = jnp.dot(x_ref[...], w1_ref[...], preferred_element_type=jnp.float32)
    h = jnp.maximum(h + b1_ref[...], 0.0)
    o_ref[...] = jnp.dot(h, w2_ref[...],
                         preferred_element_type=jnp.float32) + b2_ref[...]


def _pack_w(wk, p):
    cin, cout = wk.shape[1], wk.shape[2]
    w = jnp.zeros((p * cin, 9, p * cout), jnp.float32)
    for dy in range(3):
        for b in range(3):
            t = dy * 3 + b
            for jin in range(p):
                for jout in range(p):
                    dx = p * (b - 1) + jin - jout
                    if -1 <= dx <= 1:
                        w = w.at[jin * cin:(jin + 1) * cin, t,
                                 jout * cout:(jout + 1) * cout].set(
                                     wk[dy * 3 + dx + 1])
    w = w.reshape(p * cin, 3, 3, p * cout).transpose(2, 0, 1, 3)
    return w.reshape(3 * p * cin, 3 * p * cout)


def _col_im2col(packed):
    n, h, u, pc = packed.shape
    xs = jnp.pad(packed, ((0, 0), (1, 1), (0, 0), (0, 0)))
    zcol = jnp.zeros((n, h + 2, 1, pc), jnp.float32)
    left = jnp.concatenate([zcol, xs[:, :, :u - 1]], axis=2)
    right = jnp.concatenate([xs[:, :, 1:], zcol], axis=2)
    x3 = jnp.concatenate([left, xs, right], axis=-1)
    return jnp.pad(x3, ((0, 0), (0, 0), (0, _round8(u) - u), (0, 0)))


def kernel(x, s0_wa, s0_ba, s0_wb, s0_bb, s0_sc, s0_sh,
           s1_wa, s1_ba, s1_wb, s1_bb, s1_sc, s1_sh,
           s2_wa, s2_ba, s2_wb, s2_bb, s2_sc, s2_sh,
           s3_wa, s3_ba, s3_wb, s3_bb, s3_sc, s3_sh,
           fc1_w, fc1_b, fc2_w, fc2_b):
    n = x.shape[0]
    stages = [
        (s0_wa, s0_ba, s0_wb, s0_bb, s0_sc, s0_sh),
        (s1_wa, s1_ba, s1_wb, s1_bb, s1_sc, s1_sh),
        (s2_wa, s2_ba, s2_wb, s2_bb, s2_sc, s2_sh),
        (s3_wa, s3_ba, s3_wb, s3_bb, s3_sc, s3_sh),
    ]

    cur = jnp.transpose(x, (0, 2, 3, 1)).astype(jnp.float32)
    H = cur.shape[1]
    for (wa, ba, wb, bb, sc, sh), p in zip(stages, _PACK):
        c = cur.shape[3]
        packed = cur.reshape(n, H, H // p, p * c)
        x3 = _col_im2col(packed)
        cur = _run_stage(x3, _pack_w(wa, p), jnp.tile(ba, (1, p)),
                         _pack_w(wb, p), jnp.tile(bb, (1, p)),
                         jnp.tile(sc, (1, p // 2)), jnp.tile(sh, (1, p // 2)),
                         H, p)
        H //= 2
        cur = cur.reshape(n, H, H, wb.shape[2])

    feats = cur.reshape(n, -1)
    out_c = fc2_w.shape[1]
    return pl.pallas_call(
        _classifier_kernel,
        out_shape=jax.ShapeDtypeStruct((n, out_c), jnp.float32),
    )(feats, fc1_w, fc1_b, fc2_w, fc2_b)


# speedup vs baseline: 12.9719x; 1.4068x over previous
import functools

import jax
import jax.numpy as jnp
from jax.experimental import pallas as pl
from jax.experimental.pallas import tpu as pltpu

_PACK = (8, 2, 2, 2)


def _round8(n):
    return (n + 7) & ~7


def _stage_kernel(H, p, x_ref, wa_ref, ba_ref, wb_ref, bb_ref, sc_ref,
                  sh_ref, o_ref, x3a_ref, x3b_ref):
    Hp = H + 2
    U = H // p
    Upt = x3a_ref.shape[1]
    cmid_p = x3b_ref.shape[2] // 3
    cout_g = o_ref.shape[3]
    cout_p = 2 * cout_g

    def im2col(x3_ref, v, c):
        x3_ref[0:1] = jnp.zeros((1, Upt, 3 * c), jnp.float32)
        x3_ref[Hp - 1:Hp] = jnp.zeros((1, Upt, 3 * c), jnp.float32)
        x3_ref[:, 0:1, :] = jnp.zeros((Hp, 1, 3 * c), jnp.float32)
        x3_ref[:, U - 1:U, :] = jnp.zeros((Hp, 1, 3 * c), jnp.float32)
        x3_ref[1:H + 1, 1:U + 1, 0:c] = v
        x3_ref[1:H + 1, 0:U, c:2 * c] = v
        x3_ref[1:H + 1, 0:U - 1, 2 * c:3 * c] = v[:, 1:U, :]

    def conv(x3_ref, w_ref, b_ref, co):
        flat = x3_ref[...].reshape(Hp * Upt, x3_ref.shape[2])
        q = jnp.dot(flat, w_ref[...], preferred_element_type=jnp.float32)
        q = q.reshape(Hp, Upt, 3 * co)
        acc = q[0:H, 0:U, 0:co]
        for dy in (1, 2):
            acc = acc + q[dy:dy + H, 0:U, dy * co:(dy + 1) * co]
        return jnp.maximum(acc + b_ref[...].reshape(1, 1, co), 0.0)

    cin_p = x_ref.shape[3]
    im2col(x3a_ref, x_ref[0], cin_p)
    y = conv(x3a_ref, wa_ref, ba_ref, cmid_p)
    im2col(x3b_ref, y, cmid_p)
    y2 = conv(x3b_ref, wb_ref, bb_ref, cout_p)

    y2r = y2.reshape(H // 2, 2, U, cout_p)
    zh = jnp.maximum(y2r[:, 0], y2r[:, 1])
    c1 = cout_p // p
    parts = []
    for k in range(p // 2):
        parts.append(jnp.maximum(zh[:, :, (2 * k) * c1:(2 * k + 1) * c1],
                                 zh[:, :, (2 * k + 1) * c1:(2 * k + 2) * c1]))
    z = parts[0] if len(parts) == 1 else jnp.concatenate(parts, axis=-1)
    o_ref[0] = z * sc_ref[...].reshape(1, 1, cout_g) + \
        sh_ref[...].reshape(1, 1, cout_g)


def _const_spec(a):
    nd = a.ndim
    return pl.BlockSpec(a.shape, lambda i, _nd=nd: (0,) * _nd)


def _run_stage(xq, wa2, ba2, wb2, bb2, sc, sh, H, p):
    n = xq.shape[0]
    u, cin_p = xq.shape[2], xq.shape[3]
    hp = H + 2
    upt = _round8(u)
    cmid_p = wa2.shape[1] // 3
    cout_g = wb2.shape[1] // 6
    h2 = H // 2

    kern = functools.partial(_stage_kernel, H, p)
    return pl.pallas_call(
        kern,
        out_shape=jax.ShapeDtypeStruct((n, h2, u, cout_g), jnp.float32),
        grid=(n,),
        in_specs=[pl.BlockSpec((1, H, u, cin_p), lambda i: (i, 0, 0, 0)),
                  _const_spec(wa2), _const_spec(ba2), _const_spec(wb2),
                  _const_spec(bb2), _const_spec(sc), _const_spec(sh)],
        out_specs=pl.BlockSpec((1, h2, u, cout_g), lambda i: (i, 0, 0, 0)),
        scratch_shapes=[pltpu.VMEM((hp, upt, 3 * cin_p), jnp.float32),
                        pltpu.VMEM((hp, upt, 3 * cmid_p), jnp.float32)],
        compiler_params=pltpu.CompilerParams(
            dimension_semantics=("parallel",),
            vmem_limit_bytes=100 * 1024 * 1024,
        ),
    )(xq, wa2, ba2, wb2, bb2, sc, sh)


def _classifier_kernel(x_ref, w1_ref, b1_ref, w2_ref, b2_ref, o_ref):
    h = jnp.dot(x_ref[...], w1_ref[...], preferred_element_type=jnp.float32)
    h = jnp.maximum(h + b1_ref[...], 0.0)
    o_ref[...] = jnp.dot(h, w2_ref[...],
                         preferred_element_type=jnp.float32) + b2_ref[...]


def _pack_w(wk, p):
    cin, cout = wk.shape[1], wk.shape[2]
    w = jnp.zeros((p * cin, 9, p * cout), jnp.float32)
    for dy in range(3):
        for b in range(3):
            t = dy * 3 + b
            for jin in range(p):
                for jout in range(p):
                    dx = p * (b - 1) + jin - jout
                    if -1 <= dx <= 1:
                        w = w.at[jin * cin:(jin + 1) * cin, t,
                                 jout * cout:(jout + 1) * cout].set(
                                     wk[dy * 3 + dx + 1])
    w = w.reshape(p * cin, 3, 3, p * cout).transpose(2, 0, 1, 3)
    return w.reshape(3 * p * cin, 3 * p * cout)


def kernel(x, s0_wa, s0_ba, s0_wb, s0_bb, s0_sc, s0_sh,
           s1_wa, s1_ba, s1_wb, s1_bb, s1_sc, s1_sh,
           s2_wa, s2_ba, s2_wb, s2_bb, s2_sc, s2_sh,
           s3_wa, s3_ba, s3_wb, s3_bb, s3_sc, s3_sh,
           fc1_w, fc1_b, fc2_w, fc2_b):
    n = x.shape[0]
    stages = [
        (s0_wa, s0_ba, s0_wb, s0_bb, s0_sc, s0_sh),
        (s1_wa, s1_ba, s1_wb, s1_bb, s1_sc, s1_sh),
        (s2_wa, s2_ba, s2_wb, s2_bb, s2_sc, s2_sh),
        (s3_wa, s3_ba, s3_wb, s3_bb, s3_sc, s3_sh),
    ]

    cur = jnp.transpose(x, (0, 2, 3, 1)).astype(jnp.float32)
    H = cur.shape[1]
    for (wa, ba, wb, bb, sc, sh), p in zip(stages, _PACK):
        c = cur.shape[3]
        packed = cur.reshape(n, H, H // p, p * c)
        cur = _run_stage(packed, _pack_w(wa, p), jnp.tile(ba, (1, p)),
                         _pack_w(wb, p), jnp.tile(bb, (1, p)),
                         jnp.tile(sc, (1, p // 2)), jnp.tile(sh, (1, p // 2)),
                         H, p)
        H //= 2
        cur = cur.reshape(n, H, H, wb.shape[2])

    feats = cur.reshape(n, -1)
    out_c = fc2_w.shape[1]
    return pl.pallas_call(
        _classifier_kernel,
        out_shape=jax.ShapeDtypeStruct((n, out_c), jnp.float32),
    )(feats, fc1_w, fc1_b, fc2_w, fc2_b)


# speedup vs baseline: 16.7507x; 1.2913x over previous
import functools

import jax
import jax.numpy as jnp
from jax.experimental import pallas as pl
from jax.experimental.pallas import tpu as pltpu

_PACK = (8, 4, 2, 2)


def _round8(n):
    return (n + 7) & ~7


def _stage(v, H, p, x3a_ref, x3b_ref, wa_ref, ba_ref, wb_ref, bb_ref,
           sc_ref, sh_ref):
    Hp = H + 2
    U = H // p
    Upt = x3a_ref.shape[1]
    cmid_p = x3b_ref.shape[2] // 3
    cout_g = sc_ref.shape[1]
    cout_p = 2 * cout_g

    def im2col(x3_ref, v, c):
        x3_ref[0:1] = jnp.zeros((1, Upt, 3 * c), jnp.float32)
        x3_ref[Hp - 1:Hp] = jnp.zeros((1, Upt, 3 * c), jnp.float32)
        x3_ref[:, 0:1, :] = jnp.zeros((Hp, 1, 3 * c), jnp.float32)
        x3_ref[:, U - 1:U, :] = jnp.zeros((Hp, 1, 3 * c), jnp.float32)
        x3_ref[1:H + 1, 1:U + 1, 0:c] = v
        x3_ref[1:H + 1, 0:U, c:2 * c] = v
        x3_ref[1:H + 1, 0:U - 1, 2 * c:3 * c] = v[:, 1:U, :]

    def conv(x3_ref, w_ref, b_ref, co):
        flat = x3_ref[...].reshape(Hp * Upt, x3_ref.shape[2])
        q = jnp.dot(flat, w_ref[...], preferred_element_type=jnp.float32)
        q = q.reshape(Hp, Upt, 3 * co)
        acc = q[0:H, 0:U, 0:co]
        for dy in (1, 2):
            acc = acc + q[dy:dy + H, 0:U, dy * co:(dy + 1) * co]
        return jnp.maximum(acc + b_ref[...].reshape(1, 1, co), 0.0)

    im2col(x3a_ref, v, v.shape[2])
    y = conv(x3a_ref, wa_ref, ba_ref, cmid_p)
    im2col(x3b_ref, y, cmid_p)
    y2 = conv(x3b_ref, wb_ref, bb_ref, cout_p)

    y2r = y2.reshape(H // 2, 2, U, cout_p)
    zh = jnp.maximum(y2r[:, 0], y2r[:, 1])
    c1 = cout_p // p
    parts = []
    for k in range(p // 2):
        parts.append(jnp.maximum(zh[:, :, (2 * k) * c1:(2 * k + 1) * c1],
                                 zh[:, :, (2 * k + 1) * c1:(2 * k + 2) * c1]))
    z = parts[0] if len(parts) == 1 else jnp.concatenate(parts, axis=-1)
    return z * sc_ref[...].reshape(1, 1, cout_g) + \
        sh_ref[...].reshape(1, 1, cout_g)


def _trunk_kernel(H0, packs, *refs):
    x_ref = refs[0]
    o_ref = refs[25]
    scratches = refs[26:]
    v = x_ref[0]
    H = H0
    for i, p in enumerate(packs):
        params = refs[1 + 6 * i:7 + 6 * i]
        z = _stage(v, H, p, scratches[2 * i], scratches[2 * i + 1], *params)
        H //= 2
        if i + 1 < len(packs):
            pn = packs[i + 1]
            if pn == p // 2:
                v = z
            else:
                pairs = [jnp.concatenate([z[:, 2 * u2:2 * u2 + 1, :],
                                          z[:, 2 * u2 + 1:2 * u2 + 2, :]],
                                         axis=-1)
                         for u2 in range(z.shape[1] // 2)]
                v = jnp.concatenate(pairs, axis=1)
    o_ref[0] = z


def _const_spec(a):
    nd = a.ndim
    return pl.BlockSpec(a.shape, lambda i, _nd=nd: (0,) * _nd)


def _run_trunk(xq, flat_params, H0):
    n = xq.shape[0]
    u0, cin_p = xq.shape[2], xq.shape[3]

    in_specs = [pl.BlockSpec((1, H0, u0, cin_p), lambda i: (i, 0, 0, 0))]
    in_specs += [_const_spec(a) for a in flat_params]

    scratch = []
    H = H0
    c = cin_p // _PACK[0]
    for i, p in enumerate(_PACK):
        wa2 = flat_params[6 * i]
        cmid_p = wa2.shape[1] // 3
        hp, upt = H + 2, _round8(H // p)
        scratch.append(pltpu.VMEM((hp, upt, 3 * p * c), jnp.float32))
        scratch.append(pltpu.VMEM((hp, upt, 3 * cmid_p), jnp.float32))
        c = flat_params[6 * i + 2].shape[1] // 6
        c = c // (p // 2)
        H //= 2

    hf = H0 // 16
    kern = functools.partial(_trunk_kernel, H0, _PACK)
    return pl.pallas_call(
        kern,
        out_shape=jax.ShapeDtypeStruct((n, hf, hf, c), jnp.float32),
        grid=(n,),
        in_specs=in_specs,
        out_specs=pl.BlockSpec((1, hf, hf, c), lambda i: (i, 0, 0, 0)),
        scratch_shapes=scratch,
        compiler_params=pltpu.CompilerParams(
            dimension_semantics=("parallel",),
            vmem_limit_bytes=100 * 1024 * 1024,
        ),
    )(xq, *flat_params)


def _classifier_kernel(x_ref, w1_ref, b1_ref, w2_ref, b2_ref, o_ref):
    h = jnp.dot(x_ref[...], w1_ref[...], preferred_element_type=jnp.float32)
    h = jnp.maximum(h + b1_ref[...], 0.0)
    o_ref[...] = jnp.dot(h, w2_ref[...],
                         preferred_element_type=jnp.float32) + b2_ref[...]


def _pack_w(wk, p):
    cin, cout = wk.shape[1], wk.shape[2]
    w = jnp.zeros((p * cin, 9, p * cout), jnp.float32)
    for dy in range(3):
        for b in range(3):
            t = dy * 3 + b
            for jin in range(p):
                for jout in range(p):
                    dx = p * (b - 1) + jin - jout
                    if -1 <= dx <= 1:
                        w = w.at[jin * cin:(jin + 1) * cin, t,
                                 jout * cout:(jout + 1) * cout].set(
                                     wk[dy * 3 + dx + 1])
    w = w.reshape(p * cin, 3, 3, p * cout).transpose(2, 0, 1, 3)
    return w.reshape(3 * p * cin, 3 * p * cout)


def kernel(x, s0_wa, s0_ba, s0_wb, s0_bb, s0_sc, s0_sh,
           s1_wa, s1_ba, s1_wb, s1_bb, s1_sc, s1_sh,
           s2_wa, s2_ba, s2_wb, s2_bb, s2_sc, s2_sh,
           s3_wa, s3_ba, s3_wb, s3_bb, s3_sc, s3_sh,
           fc1_w, fc1_b, fc2_w, fc2_b):
    n = x.shape[0]
    stages = [
        (s0_wa, s0_ba, s0_wb, s0_bb, s0_sc, s0_sh),
        (s1_wa, s1_ba, s1_wb, s1_bb, s1_sc, s1_sh),
        (s2_wa, s2_ba, s2_wb, s2_bb, s2_sc, s2_sh),
        (s3_wa, s3_ba, s3_wb, s3_bb, s3_sc, s3_sh),
    ]

    xt = jnp.transpose(x, (0, 2, 3, 1)).astype(jnp.float32)
    H0 = xt.shape[1]
    p0 = _PACK[0]
    xq = xt.reshape(n, H0, H0 // p0, p0 * xt.shape[3])

    flat_params = []
    for (wa, ba, wb, bb, sc, sh), p in zip(stages, _PACK):
        flat_params += [_pack_w(wa, p), jnp.tile(ba, (1, p)),
                        _pack_w(wb, p), jnp.tile(bb, (1, p)),
                        jnp.tile(sc, (1, p // 2)), jnp.tile(sh, (1, p // 2))]

    cur = _run_trunk(xq, flat_params, H0)
    feats = cur.reshape(n, -1)
    out_c = fc2_w.shape[1]
    return pl.pallas_call(
        _classifier_kernel,
        out_shape=jax.ShapeDtypeStruct((n, out_c), jnp.float32),
    )(feats, fc1_w, fc1_b, fc2_w, fc2_b)
